# baseline (speedup 1.0000x reference)
import functools

import jax
import jax.numpy as jnp
from jax import lax
from jax.experimental import pallas as pl
from jax.experimental.pallas import tpu as pltpu
from jax.experimental.pallas import tpu_sc as plsc

_N = 10000
_E = 320000
_D = 128
_LEAKY = 0.2

_NC = 2
_NS = 16
_NW = _NC * _NS
_EW = _E // _NW
_K = 80
_H0 = 64
_H1 = 16
_C = _EW // _K
_NP = 10112
_STRIPE = _NP // _NS
_RB = 1000
_FB = 1280


def _mm_body(x_ref, w_ref, al_ref, ar_ref, wh_ref, sl_ref, sr_ref):
    wh = lax.dot_general(x_ref[...], w_ref[...], (((1,), (1,)), ((), ())),
                         preferred_element_type=jnp.float32)
    wh_ref[...] = wh
    sl_ref[...] = lax.dot_general(wh, al_ref[...], (((1,), (0,)), ((), ())),
                                  preferred_element_type=jnp.float32)
    sr_ref[...] = lax.dot_general(wh, ar_ref[...], (((1,), (0,)), ((), ())),
                                  preferred_element_type=jnp.float32)


_mm_call = pl.pallas_call(
    _mm_body,
    grid=(_N // _RB,),
    in_specs=[
        pl.BlockSpec((_RB, _D), lambda i: (i, 0)),
        pl.BlockSpec((_D, _D), lambda i: (0, 0)),
        pl.BlockSpec((_D, 1), lambda i: (0, 0)),
        pl.BlockSpec((_D, 1), lambda i: (0, 0)),
    ],
    out_specs=[
        pl.BlockSpec((_RB, _D), lambda i: (i, 0)),
        pl.BlockSpec((_RB, 1), lambda i: (i, 0)),
        pl.BlockSpec((_RB, 1), lambda i: (i, 0)),
    ],
    out_shape=[
        jax.ShapeDtypeStruct((_N, _D), jnp.float32),
        jax.ShapeDtypeStruct((_N, 1), jnp.float32),
        jax.ShapeDtypeStruct((_N, 1), jnp.float32),
    ],
)


@functools.partial(
    pl.kernel,
    out_type=(
        jax.ShapeDtypeStruct((_NC, _NP, _D), jnp.float32),
        jax.ShapeDtypeStruct((_NW, _NP), jnp.float32),
    ),
    mesh=plsc.VectorSubcoreMesh(core_axis_name="c", subcore_axis_name="s"),
    compiler_params=pltpu.CompilerParams(needs_layout_passes=False),
    scratch_types=[
        pltpu.VMEM((_N,), jnp.float32),
        pltpu.VMEM((_N,), jnp.float32),
        pltpu.VMEM((_NP,), jnp.float32),
        pltpu.VMEM((_K,), jnp.float32),
        pltpu.VMEM((2, _K), jnp.int32),
        pltpu.VMEM((2, _H0), jnp.int32),
        pltpu.VMEM((2, _H1), jnp.int32),
        pltpu.VMEM((_K, _D), jnp.float32),
        pltpu.VMEM((_H0, _D), jnp.float32),
        pltpu.VMEM_SHARED((_NP, _D), jnp.float32),
        pltpu.SemaphoreType.DMA,
        pltpu.SemaphoreType.DMA,
        pltpu.SemaphoreType.DMA,
        pltpu.SemaphoreType.DMA,
        pltpu.SemaphoreType.DMA,
    ],
)
def _edge_kernel(wh_hbm, src_hbm, dst_hbm, sl_hbm, sr_hbm,
                 outp_hbm, sparts_hbm,
                 sl_v, sr_v, svec_v, ex_v, sk_v, d0_v, d1_v, rows_v, rows0b_v,
                 out_sh, sem_i, sem_g0, sem_g1, sem_s0, sem_s1):
    cid = lax.axis_index("c")
    sid = lax.axis_index("s")
    wid = cid * _NS + sid
    ew0 = wid * _EW

    pltpu.sync_copy(sl_hbm, sl_v)
    pltpu.sync_copy(sr_hbm, sr_v)

    zeros16 = jnp.zeros((16,), jnp.float32)

    @pl.loop(0, _NP // 16)
    def _zero_svec(i):
        svec_v[pl.ds(i * 16, 16)] = zeros16

    @pl.loop(0, _K)
    def _zero_rows(i):
        for r in range(_D // 16):
            rows_v[i, pl.ds(r * 16, 16)] = zeros16

    r0 = sid * _STRIPE
    for q in range(_STRIPE // _K):
        pltpu.sync_copy(rows_v, out_sh.at[pl.ds(r0 + q * _K, _K), :])
    rem = _STRIPE - (_STRIPE // _K) * _K
    if rem:
        pltpu.sync_copy(rows_v.at[pl.ds(0, rem), :],
                        out_sh.at[pl.ds(r0 + (_STRIPE // _K) * _K, rem), :])
    plsc.subcore_barrier()

    rows0s = (rows_v.at[pl.ds(0, _H0), :], rows0b_v.at[:, :])
    rows1 = rows_v.at[pl.ds(_H0, _H1), :]

    def idx_start(c, par):
        eb = ew0 + c * _K
        pltpu.async_copy(src_hbm.at[pl.ds(eb, _K)], sk_v.at[par], sem_i)
        pltpu.async_copy(dst_hbm.at[pl.ds(eb, _H0)], d0_v.at[par], sem_i)
        pltpu.async_copy(dst_hbm.at[pl.ds(eb + _H0, _H1)], d1_v.at[par], sem_i)

    def idx_wait(c, par):
        eb = ew0 + c * _K
        pltpu.make_async_copy(src_hbm.at[pl.ds(eb, _K)], sk_v.at[par], sem_i).wait()
        pltpu.make_async_copy(dst_hbm.at[pl.ds(eb, _H0)], d0_v.at[par], sem_i).wait()
        pltpu.make_async_copy(dst_hbm.at[pl.ds(eb + _H0, _H1)], d1_v.at[par], sem_i).wait()

    def scat0_wait(par):
        pltpu.make_async_copy(rows0s[par], out_sh.at[d0_v.at[par]], sem_s0).wait()

    def scat1_wait(par):
        pltpu.make_async_copy(rows1, out_sh.at[d1_v.at[par]], sem_s1).wait()

    def phase1(par):
        for g in range(_K // 16):
            si = sk_v[par, pl.ds(g * 16, 16)]
            if g < _H0 // 16:
                di = d0_v[par, pl.ds(g * 16, 16)]
            else:
                di = d1_v[par, pl.ds((g - _H0 // 16) * 16, 16)]
            e = plsc.load_gather(sl_v, [si]) + plsc.load_gather(sr_v, [di])
            e = jnp.where(e > 0, e, _LEAKY * e)
            ex = jnp.exp(e)
            ex_v[pl.ds(g * 16, 16)] = ex
            plsc.addupdate_scatter(svec_v, [di], ex)

    def scale(base, n, par):
        buf = rows0b_v if (base == 0 and par == 1) else rows_v
        off = 0 if base == 0 else base

        @pl.loop(0, n, unroll=8)
        def _s(i):
            exi = plsc.load_gather(ex_v, [jnp.full((16,), base + i, jnp.int32)])
            for r in range(_D // 16):
                buf[off + i, pl.ds(r * 16, 16)] = (
                    buf[off + i, pl.ds(r * 16, 16)] * exi)

    def chunk(c, par, opar, first):
        idx_wait(c, par)
        g0 = pltpu.async_copy(wh_hbm.at[sk_v.at[par, pl.ds(0, _H0)]],
                              rows0s[par], sem_g0)
        phase1(par)
        if not first:
            scat1_wait(opar)
        g1 = pltpu.async_copy(wh_hbm.at[sk_v.at[par, pl.ds(_H0, _H1)]], rows1, sem_g1)
        if not first:
            scat0_wait(opar)
        cn = jnp.minimum(c + 1, _C - 1)
        idx_start(cn, opar)
        g0.wait()
        scale(0, _H0, par)
        pltpu.async_copy(rows0s[par], out_sh.at[d0_v.at[par]], sem_s0, add=True)
        g1.wait()
        scale(_H0, _H1, par)
        pltpu.async_copy(rows1, out_sh.at[d1_v.at[par]], sem_s1, add=True)

    idx_start(0, 0)
    chunk(0, 0, 1, True)

    @pl.loop(0, (_C - 1) // 2)
    def _pair(p):
        chunk(1 + 2 * p, 1, 0, False)
        chunk(2 + 2 * p, 0, 1, False)

    scat0_wait(0)
    scat1_wait(0)
    idx_wait(_C - 1, 1)

    pltpu.sync_copy(svec_v, sparts_hbm.at[wid])
    plsc.subcore_barrier()
    pltpu.sync_copy(out_sh.at[pl.ds(r0, _STRIPE), :],
                    outp_hbm.at[cid, pl.ds(r0, _STRIPE), :])


def _fin_body(p_ref, s_ref, b_ref, o_ref):
    p = p_ref[0] + p_ref[1]
    s = jnp.sum(s_ref[...], axis=0)
    o_ref[...] = p / (s + 1e-12)[:, None] + b_ref[...]


_fin_call = pl.pallas_call(
    _fin_body,
    grid=((_N + _FB - 1) // _FB,),
    in_specs=[
        pl.BlockSpec((_NC, _FB, _D), lambda i: (0, i, 0)),
        pl.BlockSpec((_NW, _FB), lambda i: (0, i)),
        pl.BlockSpec((1, _D), lambda i: (0, 0)),
    ],
    out_specs=pl.BlockSpec((_FB, _D), lambda i: (i, 0)),
    out_shape=jax.ShapeDtypeStruct((_N, _D), jnp.float32),
)


def kernel(x, edge_index, W, a_l, a_r, b):
    Wh, sl, sr = _mm_call(x, W, a_l.reshape(_D, 1), a_r.reshape(_D, 1))
    out_parts, s_parts = _edge_kernel(
        Wh, edge_index[0], edge_index[1], sl.reshape(_N), sr.reshape(_N))
    return _fin_call(out_parts, s_parts, b.reshape(1, _D))


# speedup vs baseline: 1.0423x; 1.0423x over previous
import functools

import jax
import jax.numpy as jnp
from jax import lax
from jax.experimental import pallas as pl
from jax.experimental.pallas import tpu as pltpu
from jax.experimental.pallas import tpu_sc as plsc

_N = 10000
_E = 320000
_D = 128
_LEAKY = 0.2

_NC = 2
_NS = 16
_NW = _NC * _NS
_EW = _E // _NW
_K = 80
_H0 = 48
_H1 = 32
_C = _EW // _K
_NP = 10112
_STRIPE = _NP // _NS
_RB = 1000
_FB = 1280


def _mm_body(x_ref, w_ref, al_ref, ar_ref, wh_ref, sl_ref, sr_ref):
    wh = lax.dot_general(x_ref[...], w_ref[...], (((1,), (1,)), ((), ())),
                         preferred_element_type=jnp.float32)
    wh_ref[...] = wh
    sl_ref[...] = lax.dot_general(wh, al_ref[...], (((1,), (0,)), ((), ())),
                                  preferred_element_type=jnp.float32)
    sr_ref[...] = lax.dot_general(wh, ar_ref[...], (((1,), (0,)), ((), ())),
                                  preferred_element_type=jnp.float32)


_mm_call = pl.pallas_call(
    _mm_body,
    grid=(_N // _RB,),
    in_specs=[
        pl.BlockSpec((_RB, _D), lambda i: (i, 0)),
        pl.BlockSpec((_D, _D), lambda i: (0, 0)),
        pl.BlockSpec((_D, 1), lambda i: (0, 0)),
        pl.BlockSpec((_D, 1), lambda i: (0, 0)),
    ],
    out_specs=[
        pl.BlockSpec((_RB, _D), lambda i: (i, 0)),
        pl.BlockSpec((_RB, 1), lambda i: (i, 0)),
        pl.BlockSpec((_RB, 1), lambda i: (i, 0)),
    ],
    out_shape=[
        jax.ShapeDtypeStruct((_N, _D), jnp.float32),
        jax.ShapeDtypeStruct((_N, 1), jnp.float32),
        jax.ShapeDtypeStruct((_N, 1), jnp.float32),
    ],
)


@functools.partial(
    pl.kernel,
    out_type=(
        jax.ShapeDtypeStruct((_NC, _NP, _D), jnp.float32),
        jax.ShapeDtypeStruct((_NW, _NP), jnp.float32),
    ),
    mesh=plsc.VectorSubcoreMesh(core_axis_name="c", subcore_axis_name="s"),
    compiler_params=pltpu.CompilerParams(needs_layout_passes=False),
    scratch_types=[
        pltpu.VMEM((_N,), jnp.float32),
        pltpu.VMEM((_N,), jnp.float32),
        pltpu.VMEM((_NP,), jnp.float32),
        pltpu.VMEM((_K,), jnp.float32),
        pltpu.VMEM((2, _K), jnp.int32),
        pltpu.VMEM((2, _H0), jnp.int32),
        pltpu.VMEM((2, _H1), jnp.int32),
        pltpu.VMEM((_K, _D), jnp.float32),
        pltpu.VMEM_SHARED((_NP, _D), jnp.float32),
        pltpu.SemaphoreType.DMA,
        pltpu.SemaphoreType.DMA,
        pltpu.SemaphoreType.DMA,
        pltpu.SemaphoreType.DMA,
        pltpu.SemaphoreType.DMA,
    ],
)
def _edge_kernel(wh_hbm, src_hbm, dst_hbm, sl_hbm, sr_hbm,
                 outp_hbm, sparts_hbm,
                 sl_v, sr_v, svec_v, ex_v, sk_v, d0_v, d1_v, rows_v,
                 out_sh, sem_i, sem_g0, sem_g1, sem_s0, sem_s1):
    cid = lax.axis_index("c")
    sid = lax.axis_index("s")
    wid = cid * _NS + sid
    ew0 = wid * _EW

    pltpu.sync_copy(sl_hbm, sl_v)
    pltpu.sync_copy(sr_hbm, sr_v)

    zeros16 = jnp.zeros((16,), jnp.float32)

    @pl.loop(0, _NP // 16)
    def _zero_svec(i):
        svec_v[pl.ds(i * 16, 16)] = zeros16

    @pl.loop(0, _K)
    def _zero_rows(i):
        for r in range(_D // 16):
            rows_v[i, pl.ds(r * 16, 16)] = zeros16

    r0 = sid * _STRIPE
    for q in range(_STRIPE // _K):
        pltpu.sync_copy(rows_v, out_sh.at[pl.ds(r0 + q * _K, _K), :])
    rem = _STRIPE - (_STRIPE // _K) * _K
    if rem:
        pltpu.sync_copy(rows_v.at[pl.ds(0, rem), :],
                        out_sh.at[pl.ds(r0 + (_STRIPE // _K) * _K, rem), :])
    plsc.subcore_barrier()

    rows0 = rows_v.at[pl.ds(0, _H0), :]
    rows1 = rows_v.at[pl.ds(_H0, _H1), :]

    def idx_start(c, par):
        eb = ew0 + c * _K
        pltpu.async_copy(src_hbm.at[pl.ds(eb, _K)], sk_v.at[par], sem_i)
        pltpu.async_copy(dst_hbm.at[pl.ds(eb, _H0)], d0_v.at[par], sem_i)
        pltpu.async_copy(dst_hbm.at[pl.ds(eb + _H0, _H1)], d1_v.at[par], sem_i)

    def idx_wait(c, par):
        eb = ew0 + c * _K
        pltpu.make_async_copy(src_hbm.at[pl.ds(eb, _K)], sk_v.at[par], sem_i).wait()
        pltpu.make_async_copy(dst_hbm.at[pl.ds(eb, _H0)], d0_v.at[par], sem_i).wait()
        pltpu.make_async_copy(dst_hbm.at[pl.ds(eb + _H0, _H1)], d1_v.at[par], sem_i).wait()

    def scat0_wait(par):
        pltpu.make_async_copy(rows0, out_sh.at[d0_v.at[par]], sem_s0).wait()

    def scat1_wait(par):
        pltpu.make_async_copy(rows1, out_sh.at[d1_v.at[par]], sem_s1).wait()

    def phase1(par):
        for g in range(_K // 16):
            si = sk_v[par, pl.ds(g * 16, 16)]
            if g < _H0 // 16:
                di = d0_v[par, pl.ds(g * 16, 16)]
            else:
                di = d1_v[par, pl.ds((g - _H0 // 16) * 16, 16)]
            e = plsc.load_gather(sl_v, [si]) + plsc.load_gather(sr_v, [di])
            e = jnp.where(e > 0, e, _LEAKY * e)
            ex = jnp.exp(e)
            ex_v[pl.ds(g * 16, 16)] = ex
            plsc.addupdate_scatter(svec_v, [di], ex)

    def scale(base, n):
        @pl.loop(0, n, unroll=8)
        def _s(i):
            exi = plsc.load_gather(ex_v, [jnp.full((16,), base + i, jnp.int32)])
            for r in range(_D // 16):
                rows_v[base + i, pl.ds(r * 16, 16)] = (
                    rows_v[base + i, pl.ds(r * 16, 16)] * exi)

    def chunk(c, par, opar, first):
        idx_wait(c, par)
        if not first:
            scat0_wait(opar)
        g0 = pltpu.async_copy(wh_hbm.at[sk_v.at[par, pl.ds(0, _H0)]], rows0, sem_g0)
        phase1(par)
        if not first:
            scat1_wait(opar)
        g1 = pltpu.async_copy(wh_hbm.at[sk_v.at[par, pl.ds(_H0, _H1)]], rows1, sem_g1)
        cn = jnp.minimum(c + 1, _C - 1)
        idx_start(cn, opar)
        g0.wait()
        scale(0, _H0)
        pltpu.async_copy(rows0, out_sh.at[d0_v.at[par]], sem_s0, add=True)
        g1.wait()
        scale(_H0, _H1)
        pltpu.async_copy(rows1, out_sh.at[d1_v.at[par]], sem_s1, add=True)

    idx_start(0, 0)
    chunk(0, 0, 1, True)

    @pl.loop(0, (_C - 1) // 2)
    def _pair(p):
        chunk(1 + 2 * p, 1, 0, False)
        chunk(2 + 2 * p, 0, 1, False)

    scat0_wait(0)
    scat1_wait(0)
    idx_wait(_C - 1, 1)

    pltpu.sync_copy(svec_v, sparts_hbm.at[wid])
    plsc.subcore_barrier()
    pltpu.sync_copy(out_sh.at[pl.ds(r0, _STRIPE), :],
                    outp_hbm.at[cid, pl.ds(r0, _STRIPE), :])


def _fin_body(p_ref, s_ref, b_ref, o_ref):
    p = p_ref[0] + p_ref[1]
    s = jnp.sum(s_ref[...], axis=0)
    o_ref[...] = p / (s + 1e-12)[:, None] + b_ref[...]


_fin_call = pl.pallas_call(
    _fin_body,
    grid=((_N + _FB - 1) // _FB,),
    in_specs=[
        pl.BlockSpec((_NC, _FB, _D), lambda i: (0, i, 0)),
        pl.BlockSpec((_NW, _FB), lambda i: (0, i)),
        pl.BlockSpec((1, _D), lambda i: (0, 0)),
    ],
    out_specs=pl.BlockSpec((_FB, _D), lambda i: (i, 0)),
    out_shape=jax.ShapeDtypeStruct((_N, _D), jnp.float32),
)


def kernel(x, edge_index, W, a_l, a_r, b):
    Wh, sl, sr = _mm_call(x, W, a_l.reshape(_D, 1), a_r.reshape(_D, 1))
    out_parts, s_parts = _edge_kernel(
        Wh, edge_index[0], edge_index[1], sl.reshape(_N), sr.reshape(_N))
    return _fin_call(out_parts, s_parts, b.reshape(1, _D))
